# Initial kernel scaffold; baseline (speedup 1.0000x reference)
import jax
import jax.numpy as jnp
from jax.experimental import pallas as pl


def kernel(features0, type_mask, edge_metapath_indices_0, dst_0, target_idx_0, edge_metapath_indices_1, dst_1, target_idx_1, fc0_W, fc0_b, attn_0, attn_1, fc1_W, fc1_b, fc2_W, fc_out_W, fc_out_b):
    raise NotImplementedError("write your pallas kernel here")


# speedup vs baseline: 8.9670x; 8.9670x over previous
import functools

import jax
import jax.numpy as jnp
from jax import lax
from jax.experimental import pallas as pl
from jax.experimental.pallas import tpu as pltpu

N_NODES = 100000
GRAPH_NODES = 20000
E = 160000
B = 8192
FEAT_DIM = 256
HID = 64
HEADS = 4
ATTN_VEC = 128
OUT_DIM = 64
AUG = 80
NSLOT = 2 * HEADS


def _lrelu(x):
    return jnp.where(x >= 0, x, 0.01 * x)


def _k1_body(feat_ref, w_ref, b_ref, a_ref, tf_ref, s_ref):
    tf = jnp.dot(feat_ref[...], w_ref[...].T,
                 preferred_element_type=jnp.float32) + b_ref[...]
    tf_ref[...] = tf
    s_ref[...] = _lrelu(jnp.dot(tf, a_ref[...].T,
                                preferred_element_type=jnp.float32))


def _compute_tf_s(features0, fc0_W, fc0_b, A):
    BN = 1000
    grid = (N_NODES // BN,)
    return pl.pallas_call(
        _k1_body,
        grid=grid,
        in_specs=[
            pl.BlockSpec((BN, FEAT_DIM), lambda i: (i, 0)),
            pl.BlockSpec((HID, FEAT_DIM), lambda i: (0, 0)),
            pl.BlockSpec((1, HID), lambda i: (0, 0)),
            pl.BlockSpec((NSLOT, HID), lambda i: (0, 0)),
        ],
        out_specs=[
            pl.BlockSpec((BN, HID), lambda i: (i, 0)),
            pl.BlockSpec((BN, NSLOT), lambda i: (i, 0)),
        ],
        out_shape=[
            jax.ShapeDtypeStruct((N_NODES, HID), jnp.float32),
            jax.ShapeDtypeStruct((N_NODES, NSLOT), jnp.float32),
        ],
    )(features0, fc0_W, fc0_b.reshape(1, HID), A)


def _k2_body(tf_ref, s_ref, gmax_ref, aug_ref):
    tf = tf_ref[...]
    for j in range(NSLOT):
        exn = jnp.exp(s_ref[:, j:j + 1] - gmax_ref[0, j])
        aug_ref[j, :, 0:HID] = exn * tf
        aug_ref[j, :, HID:HID + 1] = exn
        aug_ref[j, :, HID + 1:AUG] = jnp.zeros_like(aug_ref[j, :, HID + 1:AUG])


def _compute_aug(tf, s, gmax):
    BN = 1000
    grid = (N_NODES // BN,)
    return pl.pallas_call(
        _k2_body,
        grid=grid,
        in_specs=[
            pl.BlockSpec((BN, HID), lambda i: (i, 0)),
            pl.BlockSpec((BN, NSLOT), lambda i: (i, 0)),
            pl.BlockSpec((1, NSLOT), lambda i: (0, 0)),
        ],
        out_specs=pl.BlockSpec((NSLOT, BN, AUG), lambda i: (0, i, 0)),
        out_shape=jax.ShapeDtypeStruct((NSLOT, N_NODES, AUG), jnp.float32),
    )(tf, s, gmax.reshape(1, NSLOT))


def _k3_body(g_ref, w1_ref, b1_ref, o0_ref, o1_ref, sums_ref):
    step = pl.program_id(0)
    outs = []
    for mp, o_ref in ((0, o0_ref), (1, o1_ref)):
        parts = []
        for h in range(HEADS):
            num = g_ref[mp, h, :, 0:HID]
            den = g_ref[mp, h, :, HID:HID + 1]
            parts.append(jnp.where(den != 0.0, num / den, 0.0))
        x = jnp.concatenate(parts, axis=1)
        o = jnp.where(x > 0, x, jnp.exp(x) - 1.0)
        o_ref[...] = o
        outs.append(o)
    t0 = jnp.tanh(jnp.dot(outs[0], w1_ref[...].T,
                          preferred_element_type=jnp.float32) + b1_ref[...])
    t1 = jnp.tanh(jnp.dot(outs[1], w1_ref[...].T,
                          preferred_element_type=jnp.float32) + b1_ref[...])
    part = jnp.stack([jnp.sum(t0, axis=0), jnp.sum(t1, axis=0)], axis=0)

    @pl.when(step == 0)
    def _():
        sums_ref[...] = part

    @pl.when(step != 0)
    def _():
        sums_ref[...] = sums_ref[...] + part


def _tail_part1(gath, fc1_W, fc1_b):
    BB = 512
    grid = (B // BB,)
    return pl.pallas_call(
        _k3_body,
        grid=grid,
        in_specs=[
            pl.BlockSpec((2, HEADS, BB, AUG), lambda i: (0, 0, i, 0)),
            pl.BlockSpec((ATTN_VEC, HEADS * HID), lambda i: (0, 0)),
            pl.BlockSpec((1, ATTN_VEC), lambda i: (0, 0)),
        ],
        out_specs=[
            pl.BlockSpec((BB, HEADS * HID), lambda i: (i, 0)),
            pl.BlockSpec((BB, HEADS * HID), lambda i: (i, 0)),
            pl.BlockSpec((2, ATTN_VEC), lambda i: (0, 0)),
        ],
        out_shape=[
            jax.ShapeDtypeStruct((B, HEADS * HID), jnp.float32),
            jax.ShapeDtypeStruct((B, HEADS * HID), jnp.float32),
            jax.ShapeDtypeStruct((2, ATTN_VEC), jnp.float32),
        ],
    )(gath, fc1_W, fc1_b.reshape(1, ATTN_VEC))


def _k4_body(o0_ref, o1_ref, sums_ref, w2_ref, wo_ref, bo_ref,
             logits_ref, h_ref):
    mean = sums_ref[...] / float(B)
    bvec = jnp.dot(mean, w2_ref[...].T,
                   preferred_element_type=jnp.float32)
    bmax = jnp.max(bvec)
    eb = jnp.exp(bvec - bmax)
    beta = eb / jnp.sum(eb)
    h = beta[0, 0] * o0_ref[...] + beta[1, 0] * o1_ref[...]
    h_ref[...] = h
    logits_ref[...] = jnp.dot(h, wo_ref[...].T,
                              preferred_element_type=jnp.float32) + bo_ref[...]


def _tail_part2(o0, o1, sums, fc2_W, fc_out_W, fc_out_b):
    BB = 1024
    grid = (B // BB,)
    return pl.pallas_call(
        _k4_body,
        grid=grid,
        in_specs=[
            pl.BlockSpec((BB, HEADS * HID), lambda i: (i, 0)),
            pl.BlockSpec((BB, HEADS * HID), lambda i: (i, 0)),
            pl.BlockSpec((2, ATTN_VEC), lambda i: (0, 0)),
            pl.BlockSpec((1, ATTN_VEC), lambda i: (0, 0)),
            pl.BlockSpec((OUT_DIM, HEADS * HID), lambda i: (0, 0)),
            pl.BlockSpec((1, OUT_DIM), lambda i: (0, 0)),
        ],
        out_specs=[
            pl.BlockSpec((BB, OUT_DIM), lambda i: (i, 0)),
            pl.BlockSpec((BB, HEADS * HID), lambda i: (i, 0)),
        ],
        out_shape=[
            jax.ShapeDtypeStruct((B, OUT_DIM), jnp.float32),
            jax.ShapeDtypeStruct((B, HEADS * HID), jnp.float32),
        ],
    )(o0, o1, sums, fc2_W, fc_out_W, fc_out_b.reshape(1, OUT_DIM))


def _segment_middle(aug, src0, dst0, tgt0, src1, dst1, tgt1):
    gath = []
    for mp, (src, dst, tgt) in enumerate(((src0, dst0, tgt0),
                                          (src1, dst1, tgt1))):
        for h in range(HEADS):
            slot = mp * HEADS + h
            rows = aug[slot][src]
            tab = jax.ops.segment_sum(rows, dst, num_segments=GRAPH_NODES)
            gath.append(tab[tgt])
    return jnp.stack(gath, 0).reshape(2, HEADS, B, AUG)


def kernel(features0, type_mask, edge_metapath_indices_0, dst_0, target_idx_0,
           edge_metapath_indices_1, dst_1, target_idx_1,
           fc0_W, fc0_b, attn_0, attn_1, fc1_W, fc1_b, fc2_W, fc_out_W,
           fc_out_b):
    A = jnp.concatenate([attn_0[0], attn_1[0]], axis=0)
    tf, s = _compute_tf_s(features0, fc0_W, fc0_b, A)
    gmax = jnp.max(s, axis=0)
    aug = _compute_aug(tf, s, gmax)

    src0 = edge_metapath_indices_0[:, 0]
    src1 = edge_metapath_indices_1[:, 0]
    gath = _segment_middle(aug, src0, dst_0, target_idx_0,
                           src1, dst_1, target_idx_1)

    o0, o1, sums = _tail_part1(gath, fc1_W, fc1_b)
    logits, h = _tail_part2(o0, o1, sums, fc2_W, fc_out_W, fc_out_b)
    return logits, h


# speedup vs baseline: 20.1800x; 2.2505x over previous
import functools

import jax
import jax.numpy as jnp
from jax import lax
from jax.experimental import pallas as pl
from jax.experimental.pallas import tpu as pltpu
from jax.experimental.pallas import tpu_sc as plsc

N_NODES = 100000
GRAPH_NODES = 20000
E = 160000
B = 8192
FEAT_DIM = 256
HID = 64
HEADS = 4
ATTN_VEC = 128
OUT_DIM = 64
AUG = 80
NSLOT = 2 * HEADS


def _lrelu(x):
    return jnp.where(x >= 0, x, 0.01 * x)


def _k1_body(feat_ref, w_ref, b_ref, a_ref, tf_ref, s_ref):
    tf = jnp.dot(feat_ref[...], w_ref[...].T,
                 preferred_element_type=jnp.float32) + b_ref[...]
    tf_ref[...] = tf
    s_ref[...] = _lrelu(jnp.dot(tf, a_ref[...].T,
                                preferred_element_type=jnp.float32))


def _compute_tf_s(features0, fc0_W, fc0_b, A):
    BN = 1000
    grid = (N_NODES // BN,)
    return pl.pallas_call(
        _k1_body,
        grid=grid,
        in_specs=[
            pl.BlockSpec((BN, FEAT_DIM), lambda i: (i, 0)),
            pl.BlockSpec((HID, FEAT_DIM), lambda i: (0, 0)),
            pl.BlockSpec((1, HID), lambda i: (0, 0)),
            pl.BlockSpec((NSLOT, HID), lambda i: (0, 0)),
        ],
        out_specs=[
            pl.BlockSpec((BN, HID), lambda i: (i, 0)),
            pl.BlockSpec((BN, NSLOT), lambda i: (i, 0)),
        ],
        out_shape=[
            jax.ShapeDtypeStruct((N_NODES, HID), jnp.float32),
            jax.ShapeDtypeStruct((N_NODES, NSLOT), jnp.float32),
        ],
    )(features0, fc0_W, fc0_b.reshape(1, HID), A)


def _k2_body(tf_ref, s_ref, gmax_ref, aug_ref):
    tf = tf_ref[...]
    for j in range(NSLOT):
        exn = jnp.exp(s_ref[:, j:j + 1] - gmax_ref[0, j])
        aug_ref[j, :, 0:HID] = exn * tf
        aug_ref[j, :, HID:HID + 1] = exn
        aug_ref[j, :, HID + 1:AUG] = jnp.zeros_like(aug_ref[j, :, HID + 1:AUG])


def _compute_aug(tf, s, gmax):
    BN = 1000
    grid = (N_NODES // BN,)
    return pl.pallas_call(
        _k2_body,
        grid=grid,
        in_specs=[
            pl.BlockSpec((BN, HID), lambda i: (i, 0)),
            pl.BlockSpec((BN, NSLOT), lambda i: (i, 0)),
            pl.BlockSpec((1, NSLOT), lambda i: (0, 0)),
        ],
        out_specs=pl.BlockSpec((NSLOT, BN, AUG), lambda i: (0, i, 0)),
        out_shape=jax.ShapeDtypeStruct((NSLOT, N_NODES, AUG), jnp.float32),
    )(tf, s, gmax.reshape(1, NSLOT))


def _k3_body(g_ref, w1_ref, b1_ref, o0_ref, o1_ref, sums_ref):
    step = pl.program_id(0)
    outs = []
    for mp, o_ref in ((0, o0_ref), (1, o1_ref)):
        parts = []
        for h in range(HEADS):
            num = g_ref[mp, h, :, 0:HID]
            den = g_ref[mp, h, :, HID:HID + 1]
            parts.append(jnp.where(den != 0.0, num / den, 0.0))
        x = jnp.concatenate(parts, axis=1)
        o = jnp.where(x > 0, x, jnp.exp(x) - 1.0)
        o_ref[...] = o
        outs.append(o)
    t0 = jnp.tanh(jnp.dot(outs[0], w1_ref[...].T,
                          preferred_element_type=jnp.float32) + b1_ref[...])
    t1 = jnp.tanh(jnp.dot(outs[1], w1_ref[...].T,
                          preferred_element_type=jnp.float32) + b1_ref[...])
    part = jnp.stack([jnp.sum(t0, axis=0), jnp.sum(t1, axis=0)], axis=0)

    @pl.when(step == 0)
    def _():
        sums_ref[...] = part

    @pl.when(step != 0)
    def _():
        sums_ref[...] = sums_ref[...] + part


def _tail_part1(gath, fc1_W, fc1_b):
    BB = 512
    grid = (B // BB,)
    return pl.pallas_call(
        _k3_body,
        grid=grid,
        in_specs=[
            pl.BlockSpec((2, HEADS, BB, AUG), lambda i: (0, 0, i, 0)),
            pl.BlockSpec((ATTN_VEC, HEADS * HID), lambda i: (0, 0)),
            pl.BlockSpec((1, ATTN_VEC), lambda i: (0, 0)),
        ],
        out_specs=[
            pl.BlockSpec((BB, HEADS * HID), lambda i: (i, 0)),
            pl.BlockSpec((BB, HEADS * HID), lambda i: (i, 0)),
            pl.BlockSpec((2, ATTN_VEC), lambda i: (0, 0)),
        ],
        out_shape=[
            jax.ShapeDtypeStruct((B, HEADS * HID), jnp.float32),
            jax.ShapeDtypeStruct((B, HEADS * HID), jnp.float32),
            jax.ShapeDtypeStruct((2, ATTN_VEC), jnp.float32),
        ],
    )(gath, fc1_W, fc1_b.reshape(1, ATTN_VEC))


def _k4_body(o0_ref, o1_ref, sums_ref, w2_ref, wo_ref, bo_ref,
             logits_ref, h_ref):
    mean = sums_ref[...] / float(B)
    bvec = jnp.dot(mean, w2_ref[...].T,
                   preferred_element_type=jnp.float32)
    bmax = jnp.max(bvec)
    eb = jnp.exp(bvec - bmax)
    beta = eb / jnp.sum(eb)
    h = beta[0, 0] * o0_ref[...] + beta[1, 0] * o1_ref[...]
    h_ref[...] = h
    logits_ref[...] = jnp.dot(h, wo_ref[...].T,
                              preferred_element_type=jnp.float32) + bo_ref[...]


def _tail_part2(o0, o1, sums, fc2_W, fc_out_W, fc_out_b):
    BB = 1024
    grid = (B // BB,)
    return pl.pallas_call(
        _k4_body,
        grid=grid,
        in_specs=[
            pl.BlockSpec((BB, HEADS * HID), lambda i: (i, 0)),
            pl.BlockSpec((BB, HEADS * HID), lambda i: (i, 0)),
            pl.BlockSpec((2, ATTN_VEC), lambda i: (0, 0)),
            pl.BlockSpec((1, ATTN_VEC), lambda i: (0, 0)),
            pl.BlockSpec((OUT_DIM, HEADS * HID), lambda i: (0, 0)),
            pl.BlockSpec((1, OUT_DIM), lambda i: (0, 0)),
        ],
        out_specs=[
            pl.BlockSpec((BB, OUT_DIM), lambda i: (i, 0)),
            pl.BlockSpec((BB, HEADS * HID), lambda i: (i, 0)),
        ],
        out_shape=[
            jax.ShapeDtypeStruct((B, OUT_DIM), jnp.float32),
            jax.ShapeDtypeStruct((B, HEADS * HID), jnp.float32),
        ],
    )(o0, o1, sums, fc2_W, fc_out_W, fc_out_b.reshape(1, OUT_DIM))


C_EDGE = 80
C_TGT = 128
TILES = 16
EPT = E // TILES
GPT = GRAPH_NODES // TILES
BPT = B // TILES


def _sc_mid_body(aug_hbm, src0_hbm, dst0_hbm, tgt0_hbm,
                 src1_hbm, dst1_hbm, tgt1_hbm, zeros_hbm, out_hbm,
                 idx_v, dst_v, rows_v, tgt_v, outb_v, table_sh, sem):
    c = lax.axis_index("c")
    sid = lax.axis_index("s")
    for j in range(4):
        mp = j // 2
        src_hbm, dst_hbm, tgt_hbm = ((src0_hbm, dst0_hbm, tgt0_hbm)
                                     if mp == 0 else
                                     (src1_hbm, dst1_hbm, tgt1_hbm))
        off = (mp * HEADS + (j % 2)) * N_NODES + 2 * N_NODES * c
        pltpu.sync_copy(zeros_hbm, table_sh.at[pl.ds(sid * GPT, GPT)])
        plsc.subcore_barrier()

        def chunk(i, _):
            base = sid * EPT + i * C_EDGE
            pltpu.sync_copy(src_hbm.at[pl.ds(base, C_EDGE)], idx_v)
            pltpu.sync_copy(dst_hbm.at[pl.ds(base, C_EDGE)], dst_v)
            for k in range(C_EDGE // 16):
                sl = pl.ds(k * 16, 16)
                idx_v[sl] = idx_v[sl] + off
            pltpu.async_copy(aug_hbm.at[idx_v], rows_v, sem).wait()
            pltpu.sync_copy(rows_v, table_sh.at[dst_v], add=True)
            return 0

        lax.fori_loop(0, EPT // C_EDGE, chunk, 0)
        plsc.subcore_barrier()

        for k in range(BPT // C_TGT):
            tbase = sid * BPT + k * C_TGT
            pltpu.sync_copy(tgt_hbm.at[pl.ds(tbase, C_TGT)], tgt_v)
            pltpu.async_copy(table_sh.at[tgt_v], outb_v, sem).wait()
            obase = (mp * HEADS + (j % 2)) * B + 2 * B * c + tbase
            pltpu.sync_copy(outb_v, out_hbm.at[pl.ds(obase, C_TGT)])
        plsc.subcore_barrier()


def _segment_middle(aug, src0, dst0, tgt0, src1, dst1, tgt1):
    aug_flat = aug.reshape(NSLOT * N_NODES, AUG)
    zeros = jnp.zeros((GPT, AUG), jnp.float32)
    mesh = plsc.VectorSubcoreMesh(core_axis_name="c", subcore_axis_name="s")
    f = pl.kernel(
        _sc_mid_body,
        mesh=mesh,
        compiler_params=pltpu.CompilerParams(use_tc_tiling_on_sc=False),
        out_type=jax.ShapeDtypeStruct((NSLOT * B, AUG), jnp.float32),
        scratch_types=[
            pltpu.VMEM((C_EDGE,), jnp.int32),
            pltpu.VMEM((C_EDGE,), jnp.int32),
            pltpu.VMEM((C_EDGE, AUG), jnp.float32),
            pltpu.VMEM((C_TGT,), jnp.int32),
            pltpu.VMEM((C_TGT, AUG), jnp.float32),
            pltpu.VMEM_SHARED((GRAPH_NODES, AUG), jnp.float32),
            pltpu.SemaphoreType.DMA,
        ],
    )
    out = f(aug_flat, src0, dst0, tgt0, src1, dst1, tgt1, zeros)
    return out.reshape(2, HEADS, B, AUG)


def kernel(features0, type_mask, edge_metapath_indices_0, dst_0, target_idx_0,
           edge_metapath_indices_1, dst_1, target_idx_1,
           fc0_W, fc0_b, attn_0, attn_1, fc1_W, fc1_b, fc2_W, fc_out_W,
           fc_out_b):
    A = jnp.concatenate([attn_0[0], attn_1[0]], axis=0)
    tf, s = _compute_tf_s(features0, fc0_W, fc0_b, A)
    gmax = jnp.max(s, axis=0)
    aug = _compute_aug(tf, s, gmax)

    src0 = edge_metapath_indices_0[:, 0]
    src1 = edge_metapath_indices_1[:, 0]
    gath = _segment_middle(aug, src0, dst_0, target_idx_0,
                           src1, dst_1, target_idx_1)
    del type_mask

    o0, o1, sums = _tail_part1(gath, fc1_W, fc1_b)
    logits, h = _tail_part2(o0, o1, sums, fc2_W, fc_out_W, fc_out_b)
    return logits, h
